# baseline (speedup 1.0000x reference)
import functools

import jax
import jax.numpy as jnp
from jax import lax
from jax.experimental import pallas as pl
from jax.experimental.pallas import tpu as pltpu
from jax.experimental.pallas import tpu_sc as plsc

N_ATTRS = 26
VOCAB = 1000
D_MODEL = 128
BATCH = 16384
ROWS = BATCH * N_ATTRS

NC = 2
NS = 16
NW = NC * NS
RPW = ROWS // NW
CHUNK = 128
NCH = RPW // CHUNK


def _fuse_body(prior_ref, opt_ref, attr_ref, out_ref):
    i = pl.program_id(0)
    out_ref[...] = (opt_ref[...] + attr_ref[...]) * prior_ref[i, 0]


def _fused_table(attr_emb, option_embs, prior):
    return pl.pallas_call(
        _fuse_body,
        grid=(N_ATTRS,),
        in_specs=[
            pl.BlockSpec(memory_space=pltpu.SMEM),
            pl.BlockSpec((1, VOCAB, D_MODEL), lambda i: (i, 0, 0)),
            pl.BlockSpec((1, 1, D_MODEL), lambda i: (i, 0, 0)),
        ],
        out_specs=pl.BlockSpec((1, VOCAB, D_MODEL), lambda i: (i, 0, 0)),
        out_shape=jax.ShapeDtypeStruct((N_ATTRS, VOCAB, D_MODEL), jnp.float32),
    )(prior, option_embs, attr_emb.reshape(N_ATTRS, 1, D_MODEL))


def _idx_body(x_ref, out_ref):
    offs = lax.broadcasted_iota(jnp.int32, (BATCH, N_ATTRS), 1) * VOCAB
    out_ref[...] = x_ref[...] + offs


def _flat_idx(x):
    return pl.pallas_call(
        _idx_body,
        out_shape=jax.ShapeDtypeStruct((BATCH, N_ATTRS), jnp.int32),
    )(x)


SLICES = 8
SB = BATCH // SLICES
SROWS = SB * N_ATTRS
BPW = SB // NW
RPWS = BPW * N_ATTRS
CB = 16
CROWS = CB * N_ATTRS
GROWS = 104
GPC = CROWS // GROWS
NCHUNK = BPW // CB

_mesh = plsc.VectorSubcoreMesh(core_axis_name="c", subcore_axis_name="s")


@functools.partial(
    pl.kernel,
    mesh=_mesh,
    out_type=jax.ShapeDtypeStruct((SROWS, D_MODEL), jnp.float32),
    scratch_types=[
        pltpu.VMEM((RPWS,), jnp.int32),
        pltpu.VMEM((CROWS, D_MODEL), jnp.float32),
        pltpu.VMEM((CROWS, D_MODEL), jnp.float32),
        pltpu.SemaphoreType.DMA,
        pltpu.SemaphoreType.DMA,
        pltpu.SemaphoreType.DMA,
        pltpu.SemaphoreType.DMA,
    ],
)
def _gather_kernel(table_hbm, idx_hbm, out_hbm, idx_v, buf0, buf1,
                   g0, g1, s0, s1):
    wid = lax.axis_index("s") * NC + lax.axis_index("c")
    rbase = wid * RPWS
    bbase = wid * BPW
    pltpu.sync_copy(idx_hbm.at[pl.ds(rbase, RPWS)], idx_v)

    def start_gathers(c, buf, sem):
        for g in range(GPC):
            pltpu.async_copy(
                table_hbm.at[idx_v.at[pl.ds(c * CROWS + g * GROWS, GROWS)]],
                buf.at[pl.ds(g * GROWS, GROWS)], sem)

    def wait_gathers(buf, sem):
        pltpu.make_async_copy(table_hbm.at[pl.ds(0, CROWS)], buf, sem).wait()

    def start_put(c, buf, sem):
        pltpu.async_copy(buf, out_hbm.at[pl.ds(rbase + c * CROWS, CROWS)],
                         sem)

    def wait_put(c, buf, sem):
        pltpu.make_async_copy(buf,
                              out_hbm.at[pl.ds(rbase + c * CROWS, CROWS)],
                              sem).wait()

    start_gathers(0, buf0, g0)
    start_gathers(1, buf1, g1)

    def body(p, carry):
        c = 2 * p
        wait_gathers(buf0, g0)
        start_put(c, buf0, s0)
        wait_put(c, buf0, s0)
        start_gathers(c + 2, buf0, g0)
        wait_gathers(buf1, g1)
        start_put(c + 1, buf1, s1)
        wait_put(c + 1, buf1, s1)
        start_gathers(c + 3, buf1, g1)
        return carry

    lax.fori_loop(0, NCHUNK // 2 - 1, body, 0)

    c_last = NCHUNK - 2
    wait_gathers(buf0, g0)
    start_put(c_last, buf0, s0)
    wait_gathers(buf1, g1)
    start_put(c_last + 1, buf1, s1)
    wait_put(c_last, buf0, s0)
    wait_put(c_last + 1, buf1, s1)


BLK = 8


def _relayout_body(rows_ref, out_ref):
    out_ref[...] = rows_ref[...].reshape(BLK, N_ATTRS, D_MODEL)


def _relayout_init_body(big_ref, rows_ref, out_ref):
    del big_ref
    _relayout_body(rows_ref, out_ref)


def _relayout(s, big, rows):
    nblk = SB // BLK
    if s == 0:
        return pl.pallas_call(
            _relayout_body,
            grid=(nblk,),
            in_specs=[pl.BlockSpec((BLK * N_ATTRS, D_MODEL),
                                   lambda i: (i, 0))],
            out_specs=pl.BlockSpec((BLK, N_ATTRS, D_MODEL),
                                   lambda i: (i, 0, 0)),
            out_shape=jax.ShapeDtypeStruct((BATCH, N_ATTRS, D_MODEL),
                                           jnp.float32),
        )(rows)
    return pl.pallas_call(
        _relayout_init_body,
        grid=(nblk,),
        in_specs=[
            pl.BlockSpec(memory_space=pl.ANY),
            pl.BlockSpec((BLK * N_ATTRS, D_MODEL), lambda i: (i, 0)),
        ],
        out_specs=pl.BlockSpec((BLK, N_ATTRS, D_MODEL),
                               lambda i, s=s: (i + s * (SB // BLK), 0, 0)),
        out_shape=jax.ShapeDtypeStruct((BATCH, N_ATTRS, D_MODEL),
                                       jnp.float32),
        input_output_aliases={0: 0},
    )(big, rows)


def kernel(x, attr_emb, option_embs, prior):
    x = x.astype(jnp.int32)
    table = _fused_table(attr_emb, option_embs, prior).reshape(
        N_ATTRS * VOCAB, D_MODEL)
    idx = _flat_idx(x).reshape(ROWS)
    out = None
    for s in range(SLICES):
        o = _gather_kernel(
            table, lax.slice(idx, (s * SROWS,), ((s + 1) * SROWS,)))
        out = _relayout(s, out, o)
    return out


# speedup vs baseline: 3.7208x; 3.7208x over previous
import functools

import jax
import jax.numpy as jnp
from jax import lax
from jax.experimental import pallas as pl
from jax.experimental.pallas import tpu as pltpu
from jax.experimental.pallas import tpu_sc as plsc

N_ATTRS = 26
VOCAB = 1000
D_MODEL = 128
BATCH = 16384
ROWS = BATCH * N_ATTRS

NC = 2
NS = 16
NW = NC * NS


def _fuse_body(prior_ref, opt_ref, attr_ref, out_ref):
    i = pl.program_id(0)
    out_ref[...] = (opt_ref[...] + attr_ref[...]) * prior_ref[i, 0]


def _fused_table(attr_emb, option_embs, prior):
    return pl.pallas_call(
        _fuse_body,
        grid=(N_ATTRS,),
        in_specs=[
            pl.BlockSpec(memory_space=pltpu.SMEM),
            pl.BlockSpec((1, VOCAB, D_MODEL), lambda i: (i, 0, 0)),
            pl.BlockSpec((1, 1, D_MODEL), lambda i: (i, 0, 0)),
        ],
        out_specs=pl.BlockSpec((1, VOCAB, D_MODEL), lambda i: (i, 0, 0)),
        out_shape=jax.ShapeDtypeStruct((N_ATTRS, VOCAB, D_MODEL), jnp.float32),
    )(prior, option_embs, attr_emb.reshape(N_ATTRS, 1, D_MODEL))


def _idx_body(x_ref, out_ref):
    offs = lax.broadcasted_iota(jnp.int32, (BATCH, N_ATTRS), 1) * VOCAB
    out_ref[...] = x_ref[...] + offs


def _flat_idx(x):
    return pl.pallas_call(
        _idx_body,
        out_shape=jax.ShapeDtypeStruct((BATCH, N_ATTRS), jnp.int32),
    )(x)


BPW = BATCH // NW
RPW = BPW * N_ATTRS
CB = 8
CROWS = CB * N_ATTRS
GROWS = 104
GPC = CROWS // GROWS
NCHUNK = BPW // CB
NBUF = 4

_mesh = plsc.VectorSubcoreMesh(core_axis_name="c", subcore_axis_name="s")


@functools.partial(
    pl.kernel,
    mesh=_mesh,
    out_type=jax.ShapeDtypeStruct((BATCH, N_ATTRS, D_MODEL), jnp.float32),
    scratch_types=[
        pltpu.VMEM((RPW,), jnp.int32),
        [pltpu.VMEM((CROWS, D_MODEL), jnp.float32) for _ in range(NBUF)],
        [pltpu.SemaphoreType.DMA for _ in range(NBUF)],
        [pltpu.SemaphoreType.DMA for _ in range(NBUF)],
    ],
)
def _gather_kernel(table_hbm, idx_hbm, out_hbm, idx_v, bufs, gsems, ssems):
    wid = lax.axis_index("s") * NC + lax.axis_index("c")
    rbase = wid * RPW
    bbase = wid * BPW
    pltpu.sync_copy(idx_hbm.at[pl.ds(rbase, RPW)], idx_v)

    def start_gathers(c, b):
        for g in range(GPC):
            pltpu.async_copy(
                table_hbm.at[idx_v.at[pl.ds(c * CROWS + g * GROWS, GROWS)]],
                bufs[b].at[pl.ds(g * GROWS, GROWS)], gsems[b])

    def wait_gathers(b):
        pltpu.make_async_copy(table_hbm.at[pl.ds(0, CROWS)], bufs[b],
                              gsems[b]).wait()

    def start_put(c, b):
        pltpu.async_copy(bufs[b].reshape(CB, N_ATTRS, D_MODEL),
                         out_hbm.at[pl.ds(bbase + c * CB, CB)], ssems[b])

    def wait_put(c, b):
        pltpu.make_async_copy(bufs[b].reshape(CB, N_ATTRS, D_MODEL),
                              out_hbm.at[pl.ds(bbase + c * CB, CB)],
                              ssems[b]).wait()

    for b in range(NBUF):
        start_gathers(b, b)

    def body(p, carry):
        c0 = NBUF * p
        for b in range(NBUF):
            c = c0 + b
            wait_gathers(b)
            start_put(c, b)
            wait_put(c, b)
            start_gathers(c + NBUF, b)
        return carry

    lax.fori_loop(0, NCHUNK // NBUF - 1, body, 0)

    c0 = NCHUNK - NBUF
    for b in range(NBUF):
        wait_gathers(b)
        start_put(c0 + b, b)
    for b in range(NBUF):
        wait_put(c0 + b, b)


def kernel(x, attr_emb, option_embs, prior):
    x = x.astype(jnp.int32)
    table = _fused_table(attr_emb, option_embs, prior).reshape(
        N_ATTRS * VOCAB, D_MODEL)
    idx = _flat_idx(x).reshape(ROWS)
    return _gather_kernel(table, idx)


# speedup vs baseline: 3.7426x; 1.0059x over previous
import functools

import jax
import jax.numpy as jnp
from jax import lax
from jax.experimental import pallas as pl
from jax.experimental.pallas import tpu as pltpu
from jax.experimental.pallas import tpu_sc as plsc

N_ATTRS = 26
VOCAB = 1000
D_MODEL = 128
BATCH = 16384
ROWS = BATCH * N_ATTRS

NC = 2
NS = 16
NW = NC * NS


_FB = 1000


def _fuse_body(prior_ref, opt_ref, attr_ref, out_ref):
    a = pl.program_id(0)
    out_ref[...] = (opt_ref[...] + attr_ref[0]) * prior_ref[a, 0]


def _fused_table(attr_emb, option_embs, prior):
    return pl.pallas_call(
        _fuse_body,
        grid=(N_ATTRS * VOCAB // _FB,),
        in_specs=[
            pl.BlockSpec(memory_space=pltpu.SMEM),
            pl.BlockSpec((_FB, D_MODEL), lambda i: (i, 0)),
            pl.BlockSpec((1, 1, D_MODEL), lambda i: (i, 0, 0)),
        ],
        out_specs=pl.BlockSpec((_FB, D_MODEL), lambda i: (i, 0)),
        out_shape=jax.ShapeDtypeStruct((N_ATTRS * VOCAB, D_MODEL),
                                       jnp.float32),
    )(prior, option_embs.reshape(N_ATTRS * VOCAB, D_MODEL),
      attr_emb.reshape(N_ATTRS, 1, D_MODEL))


_IR = ROWS // D_MODEL


def _idx_body(x_ref, out_ref):
    j = (lax.broadcasted_iota(jnp.int32, (_IR, D_MODEL), 0) * D_MODEL
         + lax.broadcasted_iota(jnp.int32, (_IR, D_MODEL), 1))
    out_ref[...] = x_ref[...] + (j % N_ATTRS) * VOCAB


def _flat_idx(x_flat):
    return pl.pallas_call(
        _idx_body,
        out_shape=jax.ShapeDtypeStruct((_IR, D_MODEL), jnp.int32),
    )(x_flat.reshape(_IR, D_MODEL))


BPW = BATCH // NW
RPW = BPW * N_ATTRS
CB = 8
CROWS = CB * N_ATTRS
GROWS = 104
GPC = CROWS // GROWS
NCHUNK = BPW // CB
NBUF = 4

_mesh = plsc.VectorSubcoreMesh(core_axis_name="c", subcore_axis_name="s")


@functools.partial(
    pl.kernel,
    mesh=_mesh,
    out_type=jax.ShapeDtypeStruct((BATCH, N_ATTRS, D_MODEL), jnp.float32),
    scratch_types=[
        pltpu.VMEM((RPW,), jnp.int32),
        [pltpu.VMEM((CROWS, D_MODEL), jnp.float32) for _ in range(NBUF)],
        [pltpu.SemaphoreType.DMA for _ in range(NBUF)],
        [pltpu.SemaphoreType.DMA for _ in range(NBUF)],
    ],
)
def _gather_kernel(table_hbm, idx_hbm, out_hbm, idx_v, bufs, gsems, ssems):
    wid = lax.axis_index("s") * NC + lax.axis_index("c")
    rbase = wid * RPW
    bbase = wid * BPW
    pltpu.sync_copy(idx_hbm.at[pl.ds(rbase, RPW)], idx_v)

    def start_gathers(c, b):
        for g in range(GPC):
            pltpu.async_copy(
                table_hbm.at[idx_v.at[pl.ds(c * CROWS + g * GROWS, GROWS)]],
                bufs[b].at[pl.ds(g * GROWS, GROWS)], gsems[b])

    def wait_gathers(b):
        pltpu.make_async_copy(table_hbm.at[pl.ds(0, CROWS)], bufs[b],
                              gsems[b]).wait()

    def start_put(c, b):
        pltpu.async_copy(bufs[b].reshape(CB, N_ATTRS, D_MODEL),
                         out_hbm.at[pl.ds(bbase + c * CB, CB)], ssems[b])

    def wait_put(c, b):
        pltpu.make_async_copy(bufs[b].reshape(CB, N_ATTRS, D_MODEL),
                              out_hbm.at[pl.ds(bbase + c * CB, CB)],
                              ssems[b]).wait()

    for b in range(NBUF):
        start_gathers(b, b)

    def body(p, carry):
        c0 = NBUF * p
        for b in range(NBUF):
            c = c0 + b
            wait_gathers(b)
            start_put(c, b)
            wait_put(c, b)
            start_gathers(c + NBUF, b)
        return carry

    lax.fori_loop(0, NCHUNK // NBUF - 1, body, 0)

    c0 = NCHUNK - NBUF
    for b in range(NBUF):
        wait_gathers(b)
        start_put(c0 + b, b)
    for b in range(NBUF):
        wait_put(c0 + b, b)


def kernel(x, attr_emb, option_embs, prior):
    x = x.astype(jnp.int32)
    table = _fused_table(attr_emb, option_embs, prior)
    idx = _flat_idx(x.reshape(ROWS)).reshape(ROWS)
    return _gather_kernel(table, idx)


# speedup vs baseline: 3.8100x; 1.0180x over previous
import functools

import jax
import jax.numpy as jnp
from jax import lax
from jax.experimental import pallas as pl
from jax.experimental.pallas import tpu as pltpu
from jax.experimental.pallas import tpu_sc as plsc

N_ATTRS = 26
VOCAB = 1000
D_MODEL = 128
BATCH = 16384
ROWS = BATCH * N_ATTRS

NC = 2
NS = 16
NW = NC * NS


_FB = 2000


def _fuse_body(prior_ref, opt_ref, attr_ref, out_ref):
    a = 2 * pl.program_id(0)
    half = (opt_ref[pl.ds(0, VOCAB), :] + attr_ref[0]) * prior_ref[a, 0]
    half2 = (opt_ref[pl.ds(VOCAB, VOCAB), :] + attr_ref[1]) * prior_ref[a + 1, 0]
    out_ref[pl.ds(0, VOCAB), :] = half
    out_ref[pl.ds(VOCAB, VOCAB), :] = half2


def _fused_table(attr_emb, option_embs, prior):
    return pl.pallas_call(
        _fuse_body,
        grid=(N_ATTRS * VOCAB // _FB,),
        in_specs=[
            pl.BlockSpec(memory_space=pltpu.SMEM),
            pl.BlockSpec((_FB, D_MODEL), lambda i: (i, 0)),
            pl.BlockSpec((2, 1, D_MODEL), lambda i: (i, 0, 0)),
        ],
        out_specs=pl.BlockSpec((_FB, D_MODEL), lambda i: (i, 0)),
        out_shape=jax.ShapeDtypeStruct((N_ATTRS * VOCAB, D_MODEL),
                                       jnp.float32),
    )(prior, option_embs.reshape(N_ATTRS * VOCAB, D_MODEL),
      attr_emb.reshape(N_ATTRS, 1, D_MODEL))


_IR = ROWS // D_MODEL


def _idx_body(x_ref, out_ref):
    j = (lax.broadcasted_iota(jnp.int32, (_IR, D_MODEL), 0) * D_MODEL
         + lax.broadcasted_iota(jnp.int32, (_IR, D_MODEL), 1))
    out_ref[...] = x_ref[...] + (j % N_ATTRS) * VOCAB


def _flat_idx(x_flat):
    return pl.pallas_call(
        _idx_body,
        out_shape=jax.ShapeDtypeStruct((_IR, D_MODEL), jnp.int32),
    )(x_flat.reshape(_IR, D_MODEL))


BPW = BATCH // NW
RPW = BPW * N_ATTRS
CB = 4
CROWS = CB * N_ATTRS
GROWS = 104
GPC = CROWS // GROWS
NCHUNK = BPW // CB
NBUF = 8

_mesh = plsc.VectorSubcoreMesh(core_axis_name="c", subcore_axis_name="s")


@functools.partial(
    pl.kernel,
    mesh=_mesh,
    out_type=jax.ShapeDtypeStruct((BATCH, N_ATTRS, D_MODEL), jnp.float32),
    scratch_types=[
        pltpu.VMEM((RPW,), jnp.int32),
        [pltpu.VMEM((CROWS, D_MODEL), jnp.float32) for _ in range(NBUF)],
        [pltpu.SemaphoreType.DMA for _ in range(NBUF)],
        [pltpu.SemaphoreType.DMA for _ in range(NBUF)],
    ],
)
def _gather_kernel(table_hbm, idx_hbm, out_hbm, idx_v, bufs, gsems, ssems):
    wid = lax.axis_index("s") * NC + lax.axis_index("c")
    rbase = wid * RPW
    bbase = wid * BPW
    pltpu.sync_copy(idx_hbm.at[pl.ds(rbase, RPW)], idx_v)

    def start_gathers(c, b):
        for g in range(GPC):
            pltpu.async_copy(
                table_hbm.at[idx_v.at[pl.ds(c * CROWS + g * GROWS, GROWS)]],
                bufs[b].at[pl.ds(g * GROWS, GROWS)], gsems[b])

    def wait_gathers(b):
        pltpu.make_async_copy(table_hbm.at[pl.ds(0, CROWS)], bufs[b],
                              gsems[b]).wait()

    def start_put(c, b):
        pltpu.async_copy(bufs[b].reshape(CB, N_ATTRS, D_MODEL),
                         out_hbm.at[pl.ds(bbase + c * CB, CB)], ssems[b])

    def wait_put(c, b):
        pltpu.make_async_copy(bufs[b].reshape(CB, N_ATTRS, D_MODEL),
                              out_hbm.at[pl.ds(bbase + c * CB, CB)],
                              ssems[b]).wait()

    for b in range(NBUF):
        start_gathers(b, b)

    def body(p, carry):
        c0 = NBUF * p
        for b in range(NBUF):
            c = c0 + b
            wait_gathers(b)
            start_put(c, b)
            wait_put(c, b)
            start_gathers(c + NBUF, b)
        return carry

    lax.fori_loop(0, NCHUNK // NBUF - 1, body, 0)

    c0 = NCHUNK - NBUF
    for b in range(NBUF):
        wait_gathers(b)
        start_put(c0 + b, b)
    for b in range(NBUF):
        wait_put(c0 + b, b)


def kernel(x, attr_emb, option_embs, prior):
    x = x.astype(jnp.int32)
    table = _fused_table(attr_emb, option_embs, prior)
    idx = _flat_idx(x.reshape(ROWS)).reshape(ROWS)
    return _gather_kernel(table, idx)
